# baseline (speedup 1.0000x reference)
import functools

import jax
import jax.numpy as jnp
from jax import lax
from jax.experimental import pallas as pl
from jax.experimental.pallas import tpu as pltpu
from jax.experimental.pallas import tpu_sc as plsc

VOCAB_ = 100000
EMB_ = 512
HID_ = 512
ENDS_ = (0, 20000, 60000, 100000)
ROWS_PER_BLOCK = 800
SCALE_ = float(HID_) ** 0.5


def _bucket_body(t_ref, eref, pref, out):
    del t_ref
    out[...] = lax.dot_general(
        eref[...], pref[...], (((1,), (1,)), ((), ())),
        preferred_element_type=jnp.float32,
    ) * SCALE_


def _build_table(emb_0, emb_1, emb_2, proj_0, proj_1, proj_2):
    r = ROWS_PER_BLOCK
    table = None
    embs = (emb_0, emb_1, emb_2)
    projs = (proj_0, proj_1, proj_2)
    for i in range(3):
        lo, hi = ENDS_[i], ENDS_[i + 1]
        nb = (hi - lo) // r
        off = lo // r
        d = embs[i].shape[1]
        if table is None:
            in_specs = [
                pl.BlockSpec((r, d), lambda g: (g, 0)),
                pl.BlockSpec((HID_, d), lambda g: (0, 0)),
            ]
            body = lambda eref, pref, out: _bucket_body(None, eref, pref, out)
            args = (embs[i], projs[i])
            aliases = {}
        else:
            in_specs = [
                pl.BlockSpec(memory_space=pl.ANY),
                pl.BlockSpec((r, d), lambda g: (g, 0)),
                pl.BlockSpec((HID_, d), lambda g: (0, 0)),
            ]
            body = _bucket_body
            args = (table, embs[i], projs[i])
            aliases = {0: 0}
        table = pl.pallas_call(
            body,
            grid=(nb,),
            in_specs=in_specs,
            out_specs=pl.BlockSpec((r, HID_), lambda g, off=off: (g + off, 0)),
            out_shape=jax.ShapeDtypeStruct((VOCAB_, HID_), jnp.float32),
            input_output_aliases=aliases,
        )(*args)
    return table


@functools.cache
def _make_gather(seq, n_batch):
    info = plsc.get_sparse_core_info()
    nc, ns = info.num_cores, info.num_subcores
    nw = nc * ns
    b_total = seq * n_batch
    chunk = 64
    assert b_total % (nw * chunk) == 0 and n_batch % chunk == 0
    b_per_w = b_total // nw
    n_chunks = b_per_w // chunk
    mesh = plsc.VectorSubcoreMesh(core_axis_name="c", subcore_axis_name="s")

    @functools.partial(
        pl.kernel,
        mesh=mesh,
        out_type=jax.ShapeDtypeStruct((seq, n_batch, HID_), jnp.float32),
        scratch_types=[
            pltpu.VMEM((b_per_w,), jnp.int32),
            pltpu.VMEM((chunk, HID_), jnp.float32),
            pltpu.VMEM((chunk, HID_), jnp.float32),
            pltpu.SemaphoreType.DMA,
            pltpu.SemaphoreType.DMA,
        ],
    )
    def gather(table_hbm, idx_hbm, out_hbm, idx_v, rows_a, rows_b, sem_a,
               sem_b):
        wid = lax.axis_index("s") * nc + lax.axis_index("c")
        base = wid * b_per_w
        pltpu.sync_copy(idx_hbm.at[pl.ds(base, b_per_w)], idx_v)
        bufs = (rows_a, rows_b)
        sems = (sem_a, sem_b)
        copies = [None, None]
        copies[0] = pltpu.async_copy(
            table_hbm.at[idx_v.at[pl.ds(0, chunk)]], bufs[0], sems[0])
        for c in range(n_chunks):
            if c + 1 < n_chunks:
                copies[(c + 1) % 2] = pltpu.async_copy(
                    table_hbm.at[idx_v.at[pl.ds((c + 1) * chunk, chunk)]],
                    bufs[(c + 1) % 2], sems[(c + 1) % 2])
            copies[c % 2].wait()
            flat = base + c * chunk
            s = flat // n_batch
            b0 = flat % n_batch
            pltpu.sync_copy(bufs[c % 2], out_hbm.at[s, pl.ds(b0, chunk)])

    return gather


def kernel(token_ids, emb_0, emb_1, emb_2, proj_0, proj_1, proj_2):
    table = _build_table(emb_0, emb_1, emb_2, proj_0, proj_1, proj_2)
    n_batch, seq = token_ids.shape
    idx = token_ids.astype(jnp.int32).T.reshape(-1)
    out_sm = _make_gather(seq, n_batch)(table, idx)
    return jnp.transpose(out_sm, (1, 0, 2))


# speedup vs baseline: 1.2127x; 1.2127x over previous
import functools

import jax
import jax.numpy as jnp
from jax import lax
from jax.experimental import pallas as pl
from jax.experimental.pallas import tpu as pltpu
from jax.experimental.pallas import tpu_sc as plsc

VOCAB_ = 100000
EMB_ = 512
HID_ = 512
ENDS_ = (0, 20000, 60000, 100000)
ROWS_PER_BLOCK = 2000
SCALE_ = float(HID_) ** 0.5


def _bucket_body(t_ref, eref, pref, out):
    del t_ref
    out[...] = lax.dot_general(
        eref[...], pref[...], (((1,), (1,)), ((), ())),
        preferred_element_type=jnp.float32,
    ) * SCALE_


def _build_table(emb_0, emb_1, emb_2, proj_0, proj_1, proj_2):
    r = ROWS_PER_BLOCK
    table = None
    embs = (emb_0, emb_1, emb_2)
    projs = (proj_0, proj_1, proj_2)
    for i in range(3):
        lo, hi = ENDS_[i], ENDS_[i + 1]
        nb = (hi - lo) // r
        off = lo // r
        d = embs[i].shape[1]
        if table is None:
            in_specs = [
                pl.BlockSpec((r, d), lambda g: (g, 0)),
                pl.BlockSpec((HID_, d), lambda g: (0, 0)),
            ]
            body = lambda eref, pref, out: _bucket_body(None, eref, pref, out)
            args = (embs[i], projs[i])
            aliases = {}
        else:
            in_specs = [
                pl.BlockSpec(memory_space=pl.ANY),
                pl.BlockSpec((r, d), lambda g: (g, 0)),
                pl.BlockSpec((HID_, d), lambda g: (0, 0)),
            ]
            body = _bucket_body
            args = (table, embs[i], projs[i])
            aliases = {0: 0}
        table = pl.pallas_call(
            body,
            grid=(nb,),
            in_specs=in_specs,
            out_specs=pl.BlockSpec((r, HID_), lambda g, off=off: (g + off, 0)),
            out_shape=jax.ShapeDtypeStruct((VOCAB_, HID_), jnp.float32),
            input_output_aliases=aliases,
        )(*args)
    return table


@functools.cache
def _make_gather(seq, n_batch):
    info = plsc.get_sparse_core_info()
    nc, ns = info.num_cores, info.num_subcores
    nw = nc * ns
    b_total = seq * n_batch
    chunk = 64
    assert b_total % (nw * chunk) == 0 and n_batch % chunk == 0
    b_per_w = b_total // nw
    n_chunks = b_per_w // chunk
    mesh = plsc.VectorSubcoreMesh(core_axis_name="c", subcore_axis_name="s")

    @functools.partial(
        pl.kernel,
        mesh=mesh,
        out_type=jax.ShapeDtypeStruct((seq, n_batch, HID_), jnp.float32),
        scratch_types=[
            pltpu.VMEM((b_per_w,), jnp.int32),
            pltpu.VMEM((chunk, HID_), jnp.float32),
            pltpu.VMEM((chunk, HID_), jnp.float32),
            pltpu.SemaphoreType.DMA,
            pltpu.SemaphoreType.DMA,
        ],
    )
    def gather(table_hbm, idx_hbm, out_hbm, idx_v, rows_a, rows_b, sem_a,
               sem_b):
        wid = lax.axis_index("s") * nc + lax.axis_index("c")
        base = wid * b_per_w
        pltpu.sync_copy(idx_hbm.at[pl.ds(base, b_per_w)], idx_v)
        bufs = (rows_a, rows_b)
        sems = (sem_a, sem_b)
        copies = [None, None]
        copies[0] = pltpu.async_copy(
            table_hbm.at[idx_v.at[pl.ds(0, chunk)]], bufs[0], sems[0])
        for c in range(n_chunks):
            if c + 1 < n_chunks:
                copies[(c + 1) % 2] = pltpu.async_copy(
                    table_hbm.at[idx_v.at[pl.ds((c + 1) * chunk, chunk)]],
                    bufs[(c + 1) % 2], sems[(c + 1) % 2])
            copies[c % 2].wait()
            flat = base + c * chunk
            s = flat // n_batch
            b0 = flat % n_batch
            pltpu.sync_copy(bufs[c % 2], out_hbm.at[s, pl.ds(b0, chunk)])

    return gather


def kernel(token_ids, emb_0, emb_1, emb_2, proj_0, proj_1, proj_2):
    table = _build_table(emb_0, emb_1, emb_2, proj_0, proj_1, proj_2)
    n_batch, seq = token_ids.shape
    idx = token_ids.astype(jnp.int32).T.reshape(-1)
    out_sm = _make_gather(seq, n_batch)(table, idx)
    return jnp.transpose(out_sm, (1, 0, 2))


# speedup vs baseline: 1.2573x; 1.0368x over previous
import functools

import jax
import jax.numpy as jnp
from jax import lax
from jax.experimental import pallas as pl
from jax.experimental.pallas import tpu as pltpu
from jax.experimental.pallas import tpu_sc as plsc

VOCAB_ = 100000
EMB_ = 512
HID_ = 512
ENDS_ = (0, 20000, 60000, 100000)
ROWS_PER_BLOCK = 4000
SCALE_ = float(HID_) ** 0.5


def _bucket_body(t_ref, eref, pref, out):
    del t_ref
    out[...] = lax.dot_general(
        eref[...], pref[...], (((1,), (1,)), ((), ())),
        preferred_element_type=jnp.float32,
    ) * SCALE_


def _build_table(emb_0, emb_1, emb_2, proj_0, proj_1, proj_2):
    r = ROWS_PER_BLOCK
    table = None
    embs = (emb_0, emb_1, emb_2)
    projs = (proj_0, proj_1, proj_2)
    for i in range(3):
        lo, hi = ENDS_[i], ENDS_[i + 1]
        nb = (hi - lo) // r
        off = lo // r
        d = embs[i].shape[1]
        if table is None:
            in_specs = [
                pl.BlockSpec((r, d), lambda g: (g, 0)),
                pl.BlockSpec((HID_, d), lambda g: (0, 0)),
            ]
            body = lambda eref, pref, out: _bucket_body(None, eref, pref, out)
            args = (embs[i], projs[i])
            aliases = {}
        else:
            in_specs = [
                pl.BlockSpec(memory_space=pl.ANY),
                pl.BlockSpec((r, d), lambda g: (g, 0)),
                pl.BlockSpec((HID_, d), lambda g: (0, 0)),
            ]
            body = _bucket_body
            args = (table, embs[i], projs[i])
            aliases = {0: 0}
        table = pl.pallas_call(
            body,
            grid=(nb,),
            in_specs=in_specs,
            out_specs=pl.BlockSpec((r, HID_), lambda g, off=off: (g + off, 0)),
            out_shape=jax.ShapeDtypeStruct((VOCAB_, HID_), jnp.float32),
            input_output_aliases=aliases,
        )(*args)
    return table


@functools.cache
def _make_gather(seq, n_batch):
    info = plsc.get_sparse_core_info()
    nc, ns = info.num_cores, info.num_subcores
    nw = nc * ns
    b_total = seq * n_batch
    chunk = 64
    assert b_total % (nw * chunk) == 0 and n_batch % chunk == 0
    b_per_w = b_total // nw
    n_chunks = b_per_w // chunk
    mesh = plsc.VectorSubcoreMesh(core_axis_name="c", subcore_axis_name="s")

    @functools.partial(
        pl.kernel,
        mesh=mesh,
        out_type=jax.ShapeDtypeStruct((seq, n_batch, HID_), jnp.float32),
        scratch_types=[
            pltpu.VMEM((b_per_w,), jnp.int32),
            pltpu.VMEM((chunk, HID_), jnp.float32),
            pltpu.VMEM((chunk, HID_), jnp.float32),
            pltpu.SemaphoreType.DMA,
            pltpu.SemaphoreType.DMA,
        ],
    )
    def gather(table_hbm, idx_hbm, out_hbm, idx_v, rows_a, rows_b, sem_a,
               sem_b):
        wid = lax.axis_index("s") * nc + lax.axis_index("c")
        base = wid * b_per_w
        pltpu.sync_copy(idx_hbm.at[pl.ds(base, b_per_w)], idx_v)
        bufs = (rows_a, rows_b)
        sems = (sem_a, sem_b)
        copies = [None, None]
        copies[0] = pltpu.async_copy(
            table_hbm.at[idx_v.at[pl.ds(0, chunk)]], bufs[0], sems[0])
        for c in range(n_chunks):
            if c + 1 < n_chunks:
                copies[(c + 1) % 2] = pltpu.async_copy(
                    table_hbm.at[idx_v.at[pl.ds((c + 1) * chunk, chunk)]],
                    bufs[(c + 1) % 2], sems[(c + 1) % 2])
            copies[c % 2].wait()
            flat = base + c * chunk
            s = flat // n_batch
            b0 = flat % n_batch
            pltpu.sync_copy(bufs[c % 2], out_hbm.at[s, pl.ds(b0, chunk)])

    return gather


def kernel(token_ids, emb_0, emb_1, emb_2, proj_0, proj_1, proj_2):
    table = _build_table(emb_0, emb_1, emb_2, proj_0, proj_1, proj_2)
    n_batch, seq = token_ids.shape
    idx = token_ids.astype(jnp.int32).T.reshape(-1)
    out_sm = _make_gather(seq, n_batch)(table, idx)
    return jnp.transpose(out_sm, (1, 0, 2))


# speedup vs baseline: 1.2657x; 1.0067x over previous
import functools

import jax
import jax.numpy as jnp
from jax import lax
from jax.experimental import pallas as pl
from jax.experimental.pallas import tpu as pltpu
from jax.experimental.pallas import tpu_sc as plsc

VOCAB_ = 100000
EMB_ = 512
HID_ = 512
ENDS_ = (0, 20000, 60000, 100000)
ROWS_PER_BLOCK = 5000
SCALE_ = float(HID_) ** 0.5


def _bucket_body(t_ref, eref, pref, out):
    del t_ref
    out[...] = lax.dot_general(
        eref[...], pref[...], (((1,), (1,)), ((), ())),
        preferred_element_type=jnp.float32,
    ) * SCALE_


def _build_table(emb_0, emb_1, emb_2, proj_0, proj_1, proj_2):
    r = ROWS_PER_BLOCK
    table = None
    embs = (emb_0, emb_1, emb_2)
    projs = (proj_0, proj_1, proj_2)
    for i in range(3):
        lo, hi = ENDS_[i], ENDS_[i + 1]
        nb = (hi - lo) // r
        off = lo // r
        d = embs[i].shape[1]
        if table is None:
            in_specs = [
                pl.BlockSpec((r, d), lambda g: (g, 0)),
                pl.BlockSpec((HID_, d), lambda g: (0, 0)),
            ]
            body = lambda eref, pref, out: _bucket_body(None, eref, pref, out)
            args = (embs[i], projs[i])
            aliases = {}
        else:
            in_specs = [
                pl.BlockSpec(memory_space=pl.ANY),
                pl.BlockSpec((r, d), lambda g: (g, 0)),
                pl.BlockSpec((HID_, d), lambda g: (0, 0)),
            ]
            body = _bucket_body
            args = (table, embs[i], projs[i])
            aliases = {0: 0}
        table = pl.pallas_call(
            body,
            grid=(nb,),
            in_specs=in_specs,
            out_specs=pl.BlockSpec((r, HID_), lambda g, off=off: (g + off, 0)),
            out_shape=jax.ShapeDtypeStruct((VOCAB_, HID_), jnp.float32),
            input_output_aliases=aliases,
        )(*args)
    return table


@functools.cache
def _make_gather(seq, n_batch):
    info = plsc.get_sparse_core_info()
    nc, ns = info.num_cores, info.num_subcores
    nw = nc * ns
    b_total = seq * n_batch
    chunk = 64
    assert b_total % (nw * chunk) == 0 and n_batch % chunk == 0
    b_per_w = b_total // nw
    n_chunks = b_per_w // chunk
    mesh = plsc.VectorSubcoreMesh(core_axis_name="c", subcore_axis_name="s")

    @functools.partial(
        pl.kernel,
        mesh=mesh,
        out_type=jax.ShapeDtypeStruct((seq, n_batch, HID_), jnp.float32),
        scratch_types=[
            pltpu.VMEM((b_per_w,), jnp.int32),
            pltpu.VMEM((chunk, HID_), jnp.float32),
            pltpu.VMEM((chunk, HID_), jnp.float32),
            pltpu.SemaphoreType.DMA,
            pltpu.SemaphoreType.DMA,
        ],
    )
    def gather(table_hbm, idx_hbm, out_hbm, idx_v, rows_a, rows_b, sem_a,
               sem_b):
        wid = lax.axis_index("s") * nc + lax.axis_index("c")
        base = wid * b_per_w
        pltpu.sync_copy(idx_hbm.at[pl.ds(base, b_per_w)], idx_v)
        bufs = (rows_a, rows_b)
        sems = (sem_a, sem_b)
        copies = [None, None]
        copies[0] = pltpu.async_copy(
            table_hbm.at[idx_v.at[pl.ds(0, chunk)]], bufs[0], sems[0])
        for c in range(n_chunks):
            if c + 1 < n_chunks:
                copies[(c + 1) % 2] = pltpu.async_copy(
                    table_hbm.at[idx_v.at[pl.ds((c + 1) * chunk, chunk)]],
                    bufs[(c + 1) % 2], sems[(c + 1) % 2])
            copies[c % 2].wait()
            flat = base + c * chunk
            s = flat // n_batch
            b0 = flat % n_batch
            pltpu.sync_copy(bufs[c % 2], out_hbm.at[s, pl.ds(b0, chunk)])

    return gather


def kernel(token_ids, emb_0, emb_1, emb_2, proj_0, proj_1, proj_2):
    table = _build_table(emb_0, emb_1, emb_2, proj_0, proj_1, proj_2)
    n_batch, seq = token_ids.shape
    idx = token_ids.astype(jnp.int32).T.reshape(-1)
    out_sm = _make_gather(seq, n_batch)(table, idx)
    return jnp.transpose(out_sm, (1, 0, 2))


# speedup vs baseline: 1.2673x; 1.0013x over previous
import functools

import jax
import jax.numpy as jnp
from jax import lax
from jax.experimental import pallas as pl
from jax.experimental.pallas import tpu as pltpu
from jax.experimental.pallas import tpu_sc as plsc

VOCAB_ = 100000
EMB_ = 512
HID_ = 512
ENDS_ = (0, 20000, 60000, 100000)
ROWS_PER_BLOCK = 5000
SCALE_ = float(HID_) ** 0.5


def _bucket_body(t_ref, eref, pref, out):
    del t_ref
    out[...] = lax.dot_general(
        eref[...], pref[...], (((1,), (1,)), ((), ())),
        preferred_element_type=jnp.float32,
    ) * SCALE_


def _build_table(emb_0, emb_1, emb_2, proj_0, proj_1, proj_2):
    r = ROWS_PER_BLOCK
    table = None
    embs = (emb_0, emb_1, emb_2)
    projs = (proj_0, proj_1, proj_2)
    for i in range(3):
        lo, hi = ENDS_[i], ENDS_[i + 1]
        nb = (hi - lo) // r
        off = lo // r
        d = embs[i].shape[1]
        if table is None:
            in_specs = [
                pl.BlockSpec((r, d), lambda g: (g, 0)),
                pl.BlockSpec((HID_, d), lambda g: (0, 0)),
            ]
            body = lambda eref, pref, out: _bucket_body(None, eref, pref, out)
            args = (embs[i], projs[i])
            aliases = {}
        else:
            in_specs = [
                pl.BlockSpec(memory_space=pl.ANY),
                pl.BlockSpec((r, d), lambda g: (g, 0)),
                pl.BlockSpec((HID_, d), lambda g: (0, 0)),
            ]
            body = _bucket_body
            args = (table, embs[i], projs[i])
            aliases = {0: 0}
        table = pl.pallas_call(
            body,
            grid=(nb,),
            in_specs=in_specs,
            out_specs=pl.BlockSpec((r, HID_), lambda g, off=off: (g + off, 0)),
            out_shape=jax.ShapeDtypeStruct((VOCAB_, HID_), jnp.float32),
            input_output_aliases=aliases,
        )(*args)
    return table


@functools.cache
def _make_gather(seq, n_batch):
    info = plsc.get_sparse_core_info()
    nc, ns = info.num_cores, info.num_subcores
    nw = nc * ns
    b_total = seq * n_batch
    chunk = 64
    assert b_total % (nw * chunk) == 0 and n_batch % chunk == 0
    b_per_w = b_total // nw
    n_chunks = b_per_w // chunk
    mesh = plsc.VectorSubcoreMesh(core_axis_name="c", subcore_axis_name="s")

    @functools.partial(
        pl.kernel,
        mesh=mesh,
        out_type=jax.ShapeDtypeStruct((seq, n_batch, HID_), jnp.float32),
        scratch_types=[
            pltpu.VMEM((b_per_w,), jnp.int32),
            pltpu.VMEM((chunk, HID_), jnp.float32),
            pltpu.VMEM((chunk, HID_), jnp.float32),
            pltpu.SemaphoreType.DMA,
            pltpu.SemaphoreType.DMA,
            pltpu.SemaphoreType.DMA,
            pltpu.SemaphoreType.DMA,
        ],
    )
    def gather(table_hbm, idx_hbm, out_hbm, idx_v, rows_a, rows_b, gsem_a,
               gsem_b, wsem_a, wsem_b):
        wid = lax.axis_index("s") * nc + lax.axis_index("c")
        base = wid * b_per_w
        pltpu.sync_copy(idx_hbm.at[pl.ds(base, b_per_w)], idx_v)
        bufs = (rows_a, rows_b)
        gsems = (gsem_a, gsem_b)
        wsems = (wsem_a, wsem_b)

        def out_slice(c):
            flat = base + c * chunk
            return out_hbm.at[flat // n_batch, pl.ds(flat % n_batch, chunk)]

        gcp = [None, None]
        wcp = [None, None]
        gcp[0] = pltpu.async_copy(
            table_hbm.at[idx_v.at[pl.ds(0, chunk)]], bufs[0], gsems[0])
        for c in range(n_chunks):
            if c + 1 < n_chunks:
                if wcp[(c + 1) % 2] is not None:
                    wcp[(c + 1) % 2].wait()
                gcp[(c + 1) % 2] = pltpu.async_copy(
                    table_hbm.at[idx_v.at[pl.ds((c + 1) * chunk, chunk)]],
                    bufs[(c + 1) % 2], gsems[(c + 1) % 2])
            gcp[c % 2].wait()
            wcp[c % 2] = pltpu.async_copy(bufs[c % 2], out_slice(c),
                                          wsems[c % 2])
        wcp[(n_chunks - 1) % 2].wait()
        wcp[n_chunks % 2].wait()

    return gather


def kernel(token_ids, emb_0, emb_1, emb_2, proj_0, proj_1, proj_2):
    table = _build_table(emb_0, emb_1, emb_2, proj_0, proj_1, proj_2)
    n_batch, seq = token_ids.shape
    idx = token_ids.astype(jnp.int32).T.reshape(-1)
    out_sm = _make_gather(seq, n_batch)(table, idx)
    return jnp.transpose(out_sm, (1, 0, 2))
